# baseline (speedup 1.0000x reference)
import jax
import jax.numpy as jnp
from jax import lax
from jax.experimental import pallas as pl
from jax.experimental.pallas import tpu as pltpu
from jax.experimental.pallas import tpu_sc as plsc

_NE = 8
_HD = 768
_NTOK = 32768
_HALF = _NTOK // 2
_NC, _NS, _L = 2, 16, 16
_NW = _NC * _NS
_CHUNK = _HALF // _NW
_KSPLIT = 2


def _tc_logits_body(*refs):
    hs_refs = refs[:_KSPLIT]
    w_refs = refs[_KSPLIT:2 * _KSPLIT]
    b_ref = refs[2 * _KSPLIT]
    out_ref = refs[2 * _KSPLIT + 1]
    acc = b_ref[...]
    for hs_r, w_r in zip(hs_refs, w_refs):
        acc = acc + lax.dot_general(
            w_r[...], hs_r[...],
            dimension_numbers=(((1,), (1,)), ((), ())),
            preferred_element_type=jnp.float32)
    out_ref[...] = acc


def _make_logits(hs, weight, bias, n_rows, row_off):
    bt = 4096
    kb = _HD // _KSPLIT
    hs_specs = [
        pl.BlockSpec((bt, kb), lambda i, k=k, o=row_off // bt: (i + o, k))
        for k in range(_KSPLIT)
    ]
    w_specs = [
        pl.BlockSpec((_NE, kb), lambda i, k=k: (0, k)) for k in range(_KSPLIT)
    ]
    return pl.pallas_call(
        _tc_logits_body,
        grid=(n_rows // bt,),
        in_specs=hs_specs + w_specs + [pl.BlockSpec((_NE, 1), lambda i: (0, 0))],
        out_specs=pl.BlockSpec((_NE, bt), lambda i: (0, i)),
        out_shape=jax.ShapeDtypeStruct((_NE, n_rows), jnp.float32),
    )(*([hs] * _KSPLIT), *([weight] * _KSPLIT), bias.reshape(_NE, 1))


def _sc_router_body(lg_hbm, outw_hbm, oute_hbm, lg_v, ow_v, oe1_v, oe2_v,
                    sem):
    wid = lax.axis_index("s") * _NC + lax.axis_index("c")
    base = wid * _CHUNK
    nb = _HALF // _CHUNK // 2
    bidx = wid // nb
    r0 = (wid % nb) * _CHUNK
    copies = [
        pltpu.async_copy(lg_hbm.at[e, pl.ds(base, _CHUNK)],
                         lg_v.at[pl.ds(e * _CHUNK, _CHUNK)], sem)
        for e in range(_NE)
    ]
    for c in copies:
        c.wait()

    def group(j, carry):
        t = j * _L
        m1 = lg_v[pl.ds(t, _L)]
        a1 = jnp.zeros((_L,), jnp.int32)
        m2 = jnp.full((_L,), -jnp.inf, jnp.float32)
        a2 = jnp.zeros((_L,), jnp.int32)
        for e in range(1, _NE):
            le = lg_v[pl.ds(e * _CHUNK + t, _L)]
            gt1 = le > m1
            gt2 = le > m2
            m2 = jnp.where(gt1, m1, jnp.where(gt2, le, m2))
            a2 = jnp.where(gt1, a1, jnp.where(gt2, e, a2))
            m1 = jnp.where(gt1, le, m1)
            a1 = jnp.where(gt1, e, a1)
        ex = jnp.exp(m2 - m1)
        denom = ex + 1.0
        w1 = 1.0 / denom
        w2 = ex / denom
        for e in range(_NE):
            col = jnp.where(a1 == e, w1, jnp.where(a2 == e, w2, 0.0))
            ow_v[pl.ds(e * _CHUNK + t, _L)] = col
        oe1_v[pl.ds(t, _L)] = a1
        oe2_v[pl.ds(t, _L)] = a2
        return carry

    lax.fori_loop(0, _CHUNK // _L, group, 0)
    copies = [
        pltpu.async_copy(ow_v.at[pl.ds(e * _CHUNK, _CHUNK)],
                         outw_hbm.at[bidx, e, pl.ds(r0, _CHUNK)], sem)
        for e in range(_NE)
    ]
    copies.append(pltpu.async_copy(oe1_v, oute_hbm.at[0, pl.ds(base, _CHUNK)],
                                   sem))
    copies.append(pltpu.async_copy(oe2_v, oute_hbm.at[1, pl.ds(base, _CHUNK)],
                                   sem))
    for c in copies:
        c.wait()


def _tc_route_merge_body(lg_ref, wf1_ref, ef1_ref, wf_ref, ef_ref):
    bi = pl.program_id(0)

    @pl.when(bi < 2)
    def _copy():
        wf_ref[...] = wf1_ref[...]
        ef_ref[...] = ef1_ref[...]

    @pl.when(bi >= 2)
    def _route():
        lg = lg_ref[...]
        iot = lax.broadcasted_iota(jnp.int32, (_NE, 1), 0)
        m1 = jnp.max(lg, axis=0, keepdims=True)
        a1 = jnp.min(jnp.where(lg == m1, iot, _NE), axis=0, keepdims=True)
        lg2 = jnp.where(iot == a1, -jnp.inf, lg)
        m2 = jnp.max(lg2, axis=0, keepdims=True)
        a2 = jnp.min(jnp.where(lg2 == m2, iot, _NE), axis=0, keepdims=True)
        ex = jnp.exp(m2 - m1)
        denom = ex + 1.0
        w1 = 1.0 / denom
        w2 = ex / denom
        cols = jnp.where(iot == a1, w1, jnp.where(iot == a2, w2, 0.0))
        wf_ref[...] = cols[None]
        ef_ref[...] = jnp.concatenate([a1, a2], axis=0)


def kernel(hidden_states, weight, bias):
    b, s, h = hidden_states.shape
    n_tok = b * s
    hs = hidden_states.reshape(n_tok, h)

    logits1 = _make_logits(hs, weight, bias, _HALF, 0)

    router = pl.kernel(
        _sc_router_body,
        out_type=(jax.ShapeDtypeStruct((2, _NE, s), jnp.float32),
                  jax.ShapeDtypeStruct((2, _HALF), jnp.int32)),
        mesh=plsc.VectorSubcoreMesh(core_axis_name="c", subcore_axis_name="s"),
        compiler_params=pltpu.CompilerParams(needs_layout_passes=False,
                                             skip_device_barrier=True),
        scratch_types=[pltpu.VMEM((_CHUNK * _NE,), jnp.float32),
                       pltpu.VMEM((_CHUNK * _NE,), jnp.float32),
                       pltpu.VMEM((_CHUNK,), jnp.int32),
                       pltpu.VMEM((_CHUNK,), jnp.int32),
                       pltpu.SemaphoreType.DMA],
    )
    wf1, ef1 = router(logits1)

    logits2 = _make_logits(hs, weight, bias, _HALF, _HALF)

    gt = 1024
    ng = s // gt
    wf_t, ef_t = pl.pallas_call(
        _tc_route_merge_body,
        grid=(b, ng),
        in_specs=[
            pl.BlockSpec((_NE, gt),
                         lambda bi, gi: (0, jnp.maximum(bi - 2, 0) * ng + gi)),
            pl.BlockSpec((1, _NE, gt),
                         lambda bi, gi: (jnp.minimum(bi, 1), 0, gi)),
            pl.BlockSpec((2, gt),
                         lambda bi, gi: (0, jnp.minimum(bi, 1) * ng + gi)),
        ],
        out_specs=[
            pl.BlockSpec((1, _NE, gt), lambda bi, gi: (bi, 0, gi)),
            pl.BlockSpec((2, gt), lambda bi, gi: (0, bi * ng + gi)),
        ],
        out_shape=(jax.ShapeDtypeStruct((b, _NE, s), jnp.float32),
                   jax.ShapeDtypeStruct((2, n_tok), jnp.int32)),
    )(logits2, wf1, ef1)

    return wf_t.transpose(0, 2, 1), ef_t.T


# speedup vs baseline: 1.2768x; 1.2768x over previous
import jax
import jax.numpy as jnp
from jax import lax
from jax.experimental import pallas as pl
from jax.experimental.pallas import tpu as pltpu
from jax.experimental.pallas import tpu_sc as plsc

_NE = 8
_HD = 768
_NTOK = 32768
_NC, _NS, _L = 2, 16, 16
_NW = _NC * _NS
_CHUNK = _NTOK // _NW
_SUB = 128


_KSPLIT = 2


def _tc_logits_body(*refs):
    hs_refs = refs[:_KSPLIT]
    w_refs = refs[_KSPLIT:2 * _KSPLIT]
    b_ref = refs[2 * _KSPLIT]
    out_ref = refs[2 * _KSPLIT + 1]
    acc = b_ref[...]
    for hs_r, w_r in zip(hs_refs, w_refs):
        acc = acc + lax.dot_general(
            w_r[...], hs_r[...],
            dimension_numbers=(((1,), (1,)), ((), ())),
            preferred_element_type=jnp.float32)
    out_ref[...] = acc


def _sc_router_body(lg_hbm, outw_hbm, oute_hbm, lg_v, ow_v, oe1_v, oe2_v,
                    sem):
    wid = lax.axis_index("s") * _NC + lax.axis_index("c")
    base = wid * _CHUNK
    bidx = wid // (_NTOK // _CHUNK // 4)
    r0 = (wid % (_NTOK // _CHUNK // 4)) * _CHUNK
    copies = [
        pltpu.async_copy(lg_hbm.at[e, pl.ds(base, _CHUNK)],
                         lg_v.at[pl.ds(e * _CHUNK, _CHUNK)], sem)
        for e in range(_NE)
    ]
    for c in copies:
        c.wait()

    def group(j, carry):
        t = j * _L
        m1 = lg_v[pl.ds(t, _L)]
        a1 = jnp.zeros((_L,), jnp.int32)
        m2 = jnp.full((_L,), -jnp.inf, jnp.float32)
        a2 = jnp.zeros((_L,), jnp.int32)
        for e in range(1, _NE):
            le = lg_v[pl.ds(e * _CHUNK + t, _L)]
            gt1 = le > m1
            gt2 = le > m2
            m2 = jnp.where(gt1, m1, jnp.where(gt2, le, m2))
            a2 = jnp.where(gt1, a1, jnp.where(gt2, e, a2))
            m1 = jnp.where(gt1, le, m1)
            a1 = jnp.where(gt1, e, a1)
        ex = jnp.exp(m2 - m1)
        denom = ex + 1.0
        w1 = 1.0 / denom
        w2 = ex / denom
        for e in range(_NE):
            col = jnp.where(a1 == e, w1, jnp.where(a2 == e, w2, 0.0))
            ow_v[pl.ds(e * _CHUNK + t, _L)] = col
        oe1_v[pl.ds(t, _L)] = a1
        oe2_v[pl.ds(t, _L)] = a2
        return carry

    lax.fori_loop(0, _CHUNK // _L, group, 0)
    copies = [
        pltpu.async_copy(ow_v.at[pl.ds(e * _CHUNK, _CHUNK)],
                         outw_hbm.at[bidx, e, pl.ds(r0, _CHUNK)], sem)
        for e in range(_NE)
    ]
    copies.append(pltpu.async_copy(oe1_v, oute_hbm.at[0, pl.ds(base, _CHUNK)],
                                   sem))
    copies.append(pltpu.async_copy(oe2_v, oute_hbm.at[1, pl.ds(base, _CHUNK)],
                                   sem))
    for c in copies:
        c.wait()


def kernel(hidden_states, weight, bias):
    b, s, h = hidden_states.shape
    n_tok = b * s
    hs = hidden_states.reshape(n_tok, h)

    bt = 8192
    kb = h // _KSPLIT
    hs_specs = [
        pl.BlockSpec((bt, kb), lambda i, k=k: (i, k)) for k in range(_KSPLIT)
    ]
    w_specs = [
        pl.BlockSpec((_NE, kb), lambda i, k=k: (0, k)) for k in range(_KSPLIT)
    ]
    logits = pl.pallas_call(
        _tc_logits_body,
        grid=(n_tok // bt,),
        in_specs=hs_specs + w_specs + [pl.BlockSpec((_NE, 1), lambda i: (0, 0))],
        out_specs=pl.BlockSpec((_NE, bt), lambda i: (0, i)),
        out_shape=jax.ShapeDtypeStruct((_NE, n_tok), jnp.float32),
    )(*([hs] * _KSPLIT), *([weight] * _KSPLIT), bias.reshape(_NE, 1))

    router = pl.kernel(
        _sc_router_body,
        out_type=(jax.ShapeDtypeStruct((b, _NE, s), jnp.float32),
                  jax.ShapeDtypeStruct((2, n_tok), jnp.int32)),
        mesh=plsc.VectorSubcoreMesh(core_axis_name="c", subcore_axis_name="s"),
        compiler_params=pltpu.CompilerParams(needs_layout_passes=False,
                                             skip_device_barrier=True),
        scratch_types=[pltpu.VMEM((_CHUNK * _NE,), jnp.float32),
                       pltpu.VMEM((_CHUNK * _NE,), jnp.float32),
                       pltpu.VMEM((_CHUNK,), jnp.int32),
                       pltpu.VMEM((_CHUNK,), jnp.int32),
                       pltpu.SemaphoreType.DMA],
    )
    wf_t, ef_t = router(logits)
    return wf_t.transpose(0, 2, 1), ef_t.T


# speedup vs baseline: 1.3012x; 1.0191x over previous
import jax
import jax.numpy as jnp
from jax import lax
from jax.experimental import pallas as pl
from jax.experimental.pallas import tpu as pltpu
from jax.experimental.pallas import tpu_sc as plsc

_NE = 8
_HD = 768
_NTOK = 32768
_NC, _NS, _L = 2, 16, 16
_NW = _NC * _NS
_CHUNK = _NTOK // _NW
_SUB = 128


_KSPLIT = 2


def _tc_logits_body(*refs):
    hs_refs = refs[:_KSPLIT]
    w_refs = refs[_KSPLIT:2 * _KSPLIT]
    b_ref = refs[2 * _KSPLIT]
    out_ref = refs[2 * _KSPLIT + 1]
    acc = b_ref[...]
    for hs_r, w_r in zip(hs_refs, w_refs):
        acc = acc + lax.dot_general(
            w_r[...], hs_r[...],
            dimension_numbers=(((1,), (1,)), ((), ())),
            preferred_element_type=jnp.float32)
    out_ref[...] = acc


def _sc_router_body(lg_hbm, outw_hbm, oute_hbm, lg_v, ow_v, oe1_v, oe2_v,
                    sem):
    wid = lax.axis_index("s") * _NC + lax.axis_index("c")
    base = wid * _CHUNK
    bidx = wid // (_NTOK // _CHUNK // 4)
    r0 = (wid % (_NTOK // _CHUNK // 4)) * _CHUNK
    copies = [
        pltpu.async_copy(lg_hbm.at[e, pl.ds(base, _CHUNK)],
                         lg_v.at[pl.ds(e * _CHUNK, _CHUNK)], sem)
        for e in range(_NE)
    ]
    for c in copies:
        c.wait()

    def group(j, carry):
        t = j * _L
        m1 = lg_v[pl.ds(t, _L)]
        a1 = jnp.zeros((_L,), jnp.int32)
        m2 = jnp.full((_L,), -jnp.inf, jnp.float32)
        a2 = jnp.zeros((_L,), jnp.int32)
        for e in range(1, _NE):
            le = lg_v[pl.ds(e * _CHUNK + t, _L)]
            gt1 = le > m1
            gt2 = le > m2
            m2 = jnp.where(gt1, m1, jnp.where(gt2, le, m2))
            a2 = jnp.where(gt1, a1, jnp.where(gt2, e, a2))
            m1 = jnp.where(gt1, le, m1)
            a1 = jnp.where(gt1, e, a1)
        ex = jnp.exp(m2 - m1)
        denom = ex + 1.0
        w1 = 1.0 / denom
        w2 = ex / denom
        for e in range(_NE):
            col = jnp.where(a1 == e, w1, jnp.where(a2 == e, w2, 0.0))
            ow_v[pl.ds(e * _CHUNK + t, _L)] = col
        oe1_v[pl.ds(t, _L)] = a1
        oe2_v[pl.ds(t, _L)] = a2
        return carry

    lax.fori_loop(0, _CHUNK // _L, group, 0)
    copies = [
        pltpu.async_copy(ow_v.at[pl.ds(e * _CHUNK, _CHUNK)],
                         outw_hbm.at[bidx, e, pl.ds(r0, _CHUNK)], sem)
        for e in range(_NE)
    ]
    copies.append(pltpu.async_copy(oe1_v, oute_hbm.at[0, pl.ds(base, _CHUNK)],
                                   sem))
    copies.append(pltpu.async_copy(oe2_v, oute_hbm.at[1, pl.ds(base, _CHUNK)],
                                   sem))
    for c in copies:
        c.wait()


def kernel(hidden_states, weight, bias):
    b, s, h = hidden_states.shape
    n_tok = b * s
    hs = hidden_states.reshape(n_tok, h)

    bt = 2048
    kb = h // _KSPLIT
    hs_specs = [
        pl.BlockSpec((bt, kb), lambda i, k=k: (i, k)) for k in range(_KSPLIT)
    ]
    w_specs = [
        pl.BlockSpec((_NE, kb), lambda i, k=k: (0, k)) for k in range(_KSPLIT)
    ]
    logits = pl.pallas_call(
        _tc_logits_body,
        grid=(n_tok // bt,),
        in_specs=hs_specs + w_specs + [pl.BlockSpec((_NE, 1), lambda i: (0, 0))],
        out_specs=pl.BlockSpec((_NE, bt), lambda i: (0, i)),
        out_shape=jax.ShapeDtypeStruct((_NE, n_tok), jnp.float32),
    )(*([hs] * _KSPLIT), *([weight] * _KSPLIT), bias.reshape(_NE, 1))

    router = pl.kernel(
        _sc_router_body,
        out_type=(jax.ShapeDtypeStruct((b, _NE, s), jnp.float32),
                  jax.ShapeDtypeStruct((2, n_tok), jnp.int32)),
        mesh=plsc.VectorSubcoreMesh(core_axis_name="c", subcore_axis_name="s"),
        compiler_params=pltpu.CompilerParams(needs_layout_passes=False,
                                             skip_device_barrier=True),
        scratch_types=[pltpu.VMEM((_CHUNK * _NE,), jnp.float32),
                       pltpu.VMEM((_CHUNK * _NE,), jnp.float32),
                       pltpu.VMEM((_CHUNK,), jnp.int32),
                       pltpu.VMEM((_CHUNK,), jnp.int32),
                       pltpu.SemaphoreType.DMA],
    )
    wf_t, ef_t = router(logits)
    return wf_t.transpose(0, 2, 1), ef_t.T


# speedup vs baseline: 1.3071x; 1.0046x over previous
import jax
import jax.numpy as jnp
from jax import lax
from jax.experimental import pallas as pl
from jax.experimental.pallas import tpu as pltpu
from jax.experimental.pallas import tpu_sc as plsc

_NE = 8
_HD = 768
_NTOK = 32768
_NC, _NS, _L = 2, 16, 16
_NW = _NC * _NS
_CHUNK = _NTOK // _NW
_SUB = 128


_KSPLIT = 2


def _tc_logits_body(*refs):
    hs_refs = refs[:_KSPLIT]
    w_refs = refs[_KSPLIT:2 * _KSPLIT]
    b_ref = refs[2 * _KSPLIT]
    out_ref = refs[2 * _KSPLIT + 1]
    acc = b_ref[...]
    for hs_r, w_r in zip(hs_refs, w_refs):
        acc = acc + lax.dot_general(
            w_r[...], hs_r[...],
            dimension_numbers=(((1,), (1,)), ((), ())),
            preferred_element_type=jnp.float32)
    out_ref[...] = acc


def _sc_router_body(lg_hbm, outw_hbm, oute_hbm, lg_v, ow_v, oe1_v, oe2_v,
                    sem):
    wid = lax.axis_index("s") * _NC + lax.axis_index("c")
    base = wid * _CHUNK
    bidx = wid // (_NTOK // _CHUNK // 4)
    r0 = (wid % (_NTOK // _CHUNK // 4)) * _CHUNK
    copies = [
        pltpu.async_copy(lg_hbm.at[e, pl.ds(base, _CHUNK)],
                         lg_v.at[pl.ds(e * _CHUNK, _CHUNK)], sem)
        for e in range(_NE)
    ]
    for c in copies:
        c.wait()

    def group(j, carry):
        t = j * _L
        m1 = lg_v[pl.ds(t, _L)]
        a1 = jnp.zeros((_L,), jnp.int32)
        m2 = jnp.full((_L,), -jnp.inf, jnp.float32)
        a2 = jnp.zeros((_L,), jnp.int32)
        for e in range(1, _NE):
            le = lg_v[pl.ds(e * _CHUNK + t, _L)]
            gt1 = le > m1
            gt2 = le > m2
            m2 = jnp.where(gt1, m1, jnp.where(gt2, le, m2))
            a2 = jnp.where(gt1, a1, jnp.where(gt2, e, a2))
            m1 = jnp.where(gt1, le, m1)
            a1 = jnp.where(gt1, e, a1)
        ex = jnp.exp(m2 - m1)
        denom = ex + 1.0
        w1 = 1.0 / denom
        w2 = ex / denom
        for e in range(_NE):
            col = jnp.where(a1 == e, w1, jnp.where(a2 == e, w2, 0.0))
            ow_v[pl.ds(e * _CHUNK + t, _L)] = col
        oe1_v[pl.ds(t, _L)] = a1
        oe2_v[pl.ds(t, _L)] = a2
        return carry

    lax.fori_loop(0, _CHUNK // _L, group, 0)
    copies = [
        pltpu.async_copy(ow_v.at[pl.ds(e * _CHUNK, _CHUNK)],
                         outw_hbm.at[bidx, e, pl.ds(r0, _CHUNK)], sem)
        for e in range(_NE)
    ]
    copies.append(pltpu.async_copy(oe1_v, oute_hbm.at[0, pl.ds(base, _CHUNK)],
                                   sem))
    copies.append(pltpu.async_copy(oe2_v, oute_hbm.at[1, pl.ds(base, _CHUNK)],
                                   sem))
    for c in copies:
        c.wait()


def kernel(hidden_states, weight, bias):
    b, s, h = hidden_states.shape
    n_tok = b * s
    hs = hidden_states.reshape(n_tok, h)

    bt = 4096
    kb = h // _KSPLIT
    hs_specs = [
        pl.BlockSpec((bt, kb), lambda i, k=k: (i, k)) for k in range(_KSPLIT)
    ]
    w_specs = [
        pl.BlockSpec((_NE, kb), lambda i, k=k: (0, k)) for k in range(_KSPLIT)
    ]
    logits = pl.pallas_call(
        _tc_logits_body,
        grid=(n_tok // bt,),
        in_specs=hs_specs + w_specs + [pl.BlockSpec((_NE, 1), lambda i: (0, 0))],
        out_specs=pl.BlockSpec((_NE, bt), lambda i: (0, i)),
        out_shape=jax.ShapeDtypeStruct((_NE, n_tok), jnp.float32),
    )(*([hs] * _KSPLIT), *([weight] * _KSPLIT), bias.reshape(_NE, 1))

    router = pl.kernel(
        _sc_router_body,
        out_type=(jax.ShapeDtypeStruct((b, _NE, s), jnp.float32),
                  jax.ShapeDtypeStruct((2, n_tok), jnp.int32)),
        mesh=plsc.VectorSubcoreMesh(core_axis_name="c", subcore_axis_name="s"),
        compiler_params=pltpu.CompilerParams(needs_layout_passes=False,
                                             skip_device_barrier=True),
        scratch_types=[pltpu.VMEM((_CHUNK * _NE,), jnp.float32),
                       pltpu.VMEM((_CHUNK * _NE,), jnp.float32),
                       pltpu.VMEM((_CHUNK,), jnp.int32),
                       pltpu.VMEM((_CHUNK,), jnp.int32),
                       pltpu.SemaphoreType.DMA],
    )
    wf_t, ef_t = router(logits)
    return wf_t.transpose(0, 2, 1), ef_t.T
